# baseline (speedup 1.0000x reference)
import jax
import jax.numpy as jnp
from jax.experimental import pallas as pl
from jax.experimental.pallas import tpu as pltpu


def kernel(x, W):
    B = x.shape[0]
    S = x.shape[-2]
    D = x.shape[-1]

    W2 = W[: S * D].reshape(S, D)

    SBLK = 256
    n_chunks = S // SBLK

    def body(w_hbm, o_hbm, w_vmem, in_sems, out_sems):
        in_cps = []
        for c in range(n_chunks):
            cp = pltpu.make_async_copy(
                w_hbm.at[pl.ds(c * SBLK, SBLK), :],
                w_vmem.at[pl.ds(c * SBLK, SBLK), :],
                in_sems.at[c],
            )
            cp.start()
            in_cps.append(cp)
        out_cps = []
        for c in range(n_chunks):
            in_cps[c].wait()
            for b in range(B):
                cp = pltpu.make_async_copy(
                    w_vmem.at[pl.ds(c * SBLK, SBLK), :],
                    o_hbm.at[b, 0, pl.ds(c * SBLK, SBLK), :],
                    out_sems.at[c, b],
                )
                cp.start()
                out_cps.append(cp)
        for cp in out_cps:
            cp.wait()

    out = pl.pallas_call(
        body,
        in_specs=[pl.BlockSpec(memory_space=pl.ANY)],
        out_specs=pl.BlockSpec(memory_space=pl.ANY),
        out_shape=jax.ShapeDtypeStruct((B, 1, S, D), W.dtype),
        scratch_shapes=[
            pltpu.VMEM((S, D), W.dtype),
            pltpu.SemaphoreType.DMA((n_chunks,)),
            pltpu.SemaphoreType.DMA((n_chunks, B)),
        ],
    )(W2)
    return out
